# Initial kernel scaffold; baseline (speedup 1.0000x reference)
import jax
import jax.numpy as jnp
from jax.experimental import pallas as pl


def kernel(emb1, emb2):
    raise NotImplementedError("write your pallas kernel here")


# speedup vs baseline: 12.5691x; 12.5691x over previous
import jax
import jax.numpy as jnp
from jax.experimental import pallas as pl

N_ROWS = 4096
N_COLS = 4096
DIM = 64
K = 64
BR = 256


def _fused_kernel(emb1_ref, emb2_ref, adj_ref, proxy_ref):
    a = emb1_ref[...]
    b = emb2_ref[...]
    logits = jax.lax.dot_general(
        a, b, (((1,), (1,)), ((), ())), preferred_element_type=jnp.float32
    )
    x = jnp.maximum(logits, 0.0)
    xi = jax.lax.bitcast_convert_type(x, jnp.int32)

    t = jnp.zeros((BR, 1), dtype=jnp.int32)
    for bit in range(30, -1, -1):
        cand = t | (1 << bit)
        cnt = jnp.sum((xi >= cand).astype(jnp.int32), axis=1, keepdims=True)
        t = jnp.where(cnt >= K, cand, t)

    gt = xi > t
    eq = xi == t
    g = jnp.sum(gt.astype(jnp.int32), axis=1, keepdims=True)
    r = K - g

    col = jax.lax.broadcasted_iota(jnp.int32, (BR, N_COLS), 1)
    eq_i = eq.astype(jnp.int32)
    c = jnp.full((BR, 1), -1, dtype=jnp.int32)
    for bit in range(12, -1, -1):
        cand = c + (1 << bit)
        cnt = jnp.sum(eq_i * (col <= cand).astype(jnp.int32), axis=1, keepdims=True)
        c = jnp.where(cnt <= r, cand, c)
    keep = gt | (eq & (col <= c))

    m = jnp.max(x, axis=1, keepdims=True)
    e = jnp.exp(x - m)
    ek = jnp.where(keep, e, 0.0)
    s = jnp.sum(ek, axis=1, keepdims=True)
    adj_ref[...] = ek / s
    proxy_ref[...] = jnp.where(keep, jax.nn.sigmoid(x), 0.5)


@jax.jit
def kernel(emb1, emb2):
    out_shape = [
        jax.ShapeDtypeStruct((N_ROWS, N_COLS), jnp.float32),
        jax.ShapeDtypeStruct((N_ROWS, N_COLS), jnp.float32),
    ]
    adj, proxy = pl.pallas_call(
        _fused_kernel,
        grid=(N_ROWS // BR,),
        in_specs=[
            pl.BlockSpec((BR, DIM), lambda i: (i, 0)),
            pl.BlockSpec((N_COLS, DIM), lambda i: (0, 0)),
        ],
        out_specs=[
            pl.BlockSpec((BR, N_COLS), lambda i: (i, 0)),
            pl.BlockSpec((BR, N_COLS), lambda i: (i, 0)),
        ],
        out_shape=out_shape,
    )(emb1, emb2)
    return (adj, proxy)


# speedup vs baseline: 13.9780x; 1.1121x over previous
import jax
import jax.numpy as jnp
from jax.experimental import pallas as pl

N_ROWS = 4096
N_COLS = 4096
DIM = 64
K = 64
BR = 256
NCHUNK = 32


def _fused_kernel(emb1_ref, emb2_ref, adj_ref, proxy_ref):
    a = emb1_ref[...]
    b = emb2_ref[...]
    logits = jax.lax.dot_general(
        a, b, (((1,), (1,)), ((), ())), preferred_element_type=jnp.float32
    )
    x = jnp.maximum(logits, 0.0)
    xi = jax.lax.bitcast_convert_type(x, jnp.int32)

    fold = xi[:, :128]
    for kchunk in range(1, NCHUNK):
        fold = jnp.maximum(fold, xi[:, kchunk * 128 : (kchunk + 1) * 128])
    lo0 = jnp.min(fold, axis=1, keepdims=True)
    hi0 = jnp.max(fold, axis=1, keepdims=True) + 1

    def count_ge(v):
        return jnp.sum((xi >= v).astype(jnp.int32), axis=1, keepdims=True)

    def val_cond(carry):
        lo, hi = carry
        return jnp.any(hi - lo > 1)

    def val_body(carry):
        lo, hi = carry
        mid = lo + ((hi - lo) >> 1)
        big = count_ge(mid) >= K
        return jnp.where(big, mid, lo), jnp.where(big, hi, mid)

    t, _ = jax.lax.while_loop(val_cond, val_body, (lo0, hi0))

    ge = xi >= t
    gt = xi > t
    eq = ge & ~gt
    n_ge = jnp.sum(ge.astype(jnp.int32), axis=1, keepdims=True)
    g = jnp.sum(gt.astype(jnp.int32), axis=1, keepdims=True)
    r = K - g

    col = jax.lax.broadcasted_iota(jnp.int32, (BR, N_COLS), 1)
    eq_i = eq.astype(jnp.int32)
    surplus = n_ge > K
    clo0 = jnp.where(surplus, jnp.full_like(n_ge, -1), jnp.full_like(n_ge, N_COLS - 1))
    chi0 = jnp.full_like(n_ge, N_COLS)

    def col_cond(carry):
        clo, chi = carry
        return jnp.any(chi - clo > 1)

    def col_body(carry):
        clo, chi = carry
        mid = clo + ((chi - clo) >> 1)
        cnt = jnp.sum(eq_i * (col <= mid).astype(jnp.int32), axis=1, keepdims=True)
        small = cnt <= r
        return jnp.where(small, mid, clo), jnp.where(small, chi, mid)

    c, _ = jax.lax.while_loop(col_cond, col_body, (clo0, chi0))
    keep = gt | (eq & (col <= c))

    m = jnp.max(x, axis=1, keepdims=True)
    e = jnp.exp(x - m)
    ek = jnp.where(keep, e, 0.0)
    inv_s = 1.0 / jnp.sum(ek, axis=1, keepdims=True)
    adj_ref[...] = ek * inv_s
    proxy_ref[...] = jnp.where(keep, jax.nn.sigmoid(x), 0.5)


@jax.jit
def kernel(emb1, emb2):
    out_shape = [
        jax.ShapeDtypeStruct((N_ROWS, N_COLS), jnp.float32),
        jax.ShapeDtypeStruct((N_ROWS, N_COLS), jnp.float32),
    ]
    adj, proxy = pl.pallas_call(
        _fused_kernel,
        grid=(N_ROWS // BR,),
        in_specs=[
            pl.BlockSpec((BR, DIM), lambda i: (i, 0)),
            pl.BlockSpec((N_COLS, DIM), lambda i: (0, 0)),
        ],
        out_specs=[
            pl.BlockSpec((BR, N_COLS), lambda i: (i, 0)),
            pl.BlockSpec((BR, N_COLS), lambda i: (i, 0)),
        ],
        out_shape=out_shape,
    )(emb1, emb2)
    return (adj, proxy)


# speedup vs baseline: 16.8669x; 1.2067x over previous
import jax
import jax.numpy as jnp
from jax.experimental import pallas as pl

N_ROWS = 4096
N_COLS = 4096
DIM = 64
K = 64
BR = 256
NCHUNK = 32
UNROLL = 25


def _fused_kernel(emb1_ref, emb2_ref, adj_ref, proxy_ref):
    a = emb1_ref[...]
    b = emb2_ref[...]
    logits = jax.lax.dot_general(
        a, b, (((1,), (1,)), ((), ())), preferred_element_type=jnp.float32
    )
    x = jnp.maximum(logits, 0.0)
    xi = jax.lax.bitcast_convert_type(x, jnp.int32)

    fold = xi[:, :128]
    for kchunk in range(1, NCHUNK):
        fold = jnp.maximum(fold, xi[:, kchunk * 128 : (kchunk + 1) * 128])
    lo0 = jnp.min(fold, axis=1, keepdims=True)
    hi0 = jnp.max(fold, axis=1, keepdims=True) + 1

    def count_ge(v):
        return jnp.sum((xi >= v).astype(jnp.int32), axis=1, keepdims=True)

    def step(carry):
        lo, hi, cl, ch = carry
        mid = lo + ((hi - lo) >> 1)
        cnt = count_ge(mid)
        big = cnt >= K
        return (
            jnp.where(big, mid, lo),
            jnp.where(big, hi, mid),
            jnp.where(big, cnt, cl),
            jnp.where(big, ch, cnt),
        )

    carry = (lo0, hi0, count_ge(lo0), jnp.zeros_like(lo0))
    for _ in range(UNROLL):
        carry = step(carry)
    carry = jax.lax.while_loop(
        lambda cy: jnp.any(cy[1] - cy[0] > 1), step, carry
    )
    t, thi, n_ge, g = carry

    gt = xi >= thi
    ge = xi >= t
    eq = ge & ~gt
    r = K - g

    col = jax.lax.broadcasted_iota(jnp.int32, (BR, N_COLS), 1)
    eq_i = eq.astype(jnp.int32)
    surplus = n_ge > K
    clo0 = jnp.where(surplus, jnp.full_like(n_ge, -1), jnp.full_like(n_ge, N_COLS - 1))
    chi0 = jnp.full_like(n_ge, N_COLS)

    def col_body(carry):
        clo, chi = carry
        mid = clo + ((chi - clo) >> 1)
        cnt = jnp.sum(eq_i * (col <= mid).astype(jnp.int32), axis=1, keepdims=True)
        small = cnt <= r
        return jnp.where(small, mid, clo), jnp.where(small, chi, mid)

    c, _ = jax.lax.while_loop(
        lambda cy: jnp.any(cy[1] - cy[0] > 1), col_body, (clo0, chi0)
    )
    keep = gt | (eq & (col <= c))

    m = jnp.max(x, axis=1, keepdims=True)
    e = jnp.exp(x - m)
    ek = jnp.where(keep, e, 0.0)
    inv_s = 1.0 / jnp.sum(ek, axis=1, keepdims=True)
    adj_ref[...] = ek * inv_s
    proxy_ref[...] = jnp.where(keep, jax.nn.sigmoid(x), 0.5)


@jax.jit
def kernel(emb1, emb2):
    out_shape = [
        jax.ShapeDtypeStruct((N_ROWS, N_COLS), jnp.float32),
        jax.ShapeDtypeStruct((N_ROWS, N_COLS), jnp.float32),
    ]
    adj, proxy = pl.pallas_call(
        _fused_kernel,
        grid=(N_ROWS // BR,),
        in_specs=[
            pl.BlockSpec((BR, DIM), lambda i: (i, 0)),
            pl.BlockSpec((N_COLS, DIM), lambda i: (0, 0)),
        ],
        out_specs=[
            pl.BlockSpec((BR, N_COLS), lambda i: (i, 0)),
            pl.BlockSpec((BR, N_COLS), lambda i: (i, 0)),
        ],
        out_shape=out_shape,
    )(emb1, emb2)
    return (adj, proxy)


# speedup vs baseline: 22.2087x; 1.3167x over previous
import jax
import jax.numpy as jnp
from jax.experimental import pallas as pl

N_ROWS = 4096
N_COLS = 4096
DIM = 64
K = 64
BR = 256
NCHUNK = 32
UNROLL = 18


def _fused_kernel(emb1_ref, emb2_ref, adj_ref, proxy_ref):
    a = emb1_ref[...]
    b = emb2_ref[...]
    logits = jax.lax.dot_general(
        a, b, (((1,), (1,)), ((), ())), preferred_element_type=jnp.float32
    )
    x = jnp.maximum(logits, 0.0)
    xi = jax.lax.bitcast_convert_type(x, jnp.int32)

    fold = xi[:, :128]
    for kchunk in range(1, NCHUNK):
        fold = jnp.maximum(fold, xi[:, kchunk * 128 : (kchunk + 1) * 128])
    lo0 = jnp.min(fold, axis=1, keepdims=True)
    hi0 = jnp.max(fold, axis=1, keepdims=True) + 1

    def count_ge(v):
        return jnp.sum((xi >= v).astype(jnp.int32), axis=1, keepdims=True)

    def step(carry):
        lo, hi, cl, ch = carry
        mid = lo + ((hi - lo) >> 1)
        cnt = count_ge(mid)
        big = cnt >= K
        return (
            jnp.where(big, mid, lo),
            jnp.where(big, hi, mid),
            jnp.where(big, cnt, cl),
            jnp.where(big, ch, cnt),
        )

    def not_done(cy):
        lo, hi, cl, _ = cy
        return jnp.any((cl != K) & (hi - lo > 1))

    carry = (lo0, hi0, count_ge(lo0), jnp.zeros((BR, 1), dtype=jnp.int32))
    for _ in range(UNROLL):
        carry = step(carry)
    carry = jax.lax.while_loop(not_done, step, carry)
    t, thi, n_ge, g = carry

    ge = xi >= t
    gt = xi >= thi
    r = K - g

    col = jax.lax.broadcasted_iota(jnp.int32, (BR, N_COLS), 1)
    surplus = n_ge > K
    clo0 = jnp.where(surplus, -1, N_COLS - 1).astype(jnp.int32)
    chi0 = jnp.full((BR, 1), N_COLS, dtype=jnp.int32)

    def col_body(carry):
        clo, chi = carry
        mid = clo + ((chi - clo) >> 1)
        inb = (ge & ~gt & (col <= mid)).astype(jnp.int32)
        cnt = jnp.sum(inb, axis=1, keepdims=True)
        small = cnt <= r
        return jnp.where(small, mid, clo), jnp.where(small, chi, mid)

    c, _ = jax.lax.while_loop(
        lambda cy: jnp.any(cy[1] - cy[0] > 1), col_body, (clo0, chi0)
    )
    keep = ge & (gt | (col <= c))

    m = jax.lax.bitcast_convert_type(hi0 - 1, jnp.float32)
    e = jnp.exp(x - m)
    ek = jnp.where(keep, e, 0.0)
    s = jnp.sum(ek, axis=1, keepdims=True)
    adj_ref[...] = ek * (1.0 / s)
    proxy_ref[...] = jnp.where(keep, jax.nn.sigmoid(x), 0.5)


@jax.jit
def kernel(emb1, emb2):
    out_shape = [
        jax.ShapeDtypeStruct((N_ROWS, N_COLS), jnp.float32),
        jax.ShapeDtypeStruct((N_ROWS, N_COLS), jnp.float32),
    ]
    adj, proxy = pl.pallas_call(
        _fused_kernel,
        grid=(N_ROWS // BR,),
        in_specs=[
            pl.BlockSpec((BR, DIM), lambda i: (i, 0)),
            pl.BlockSpec((N_COLS, DIM), lambda i: (0, 0)),
        ],
        out_specs=[
            pl.BlockSpec((BR, N_COLS), lambda i: (i, 0)),
            pl.BlockSpec((BR, N_COLS), lambda i: (i, 0)),
        ],
        out_shape=out_shape,
    )(emb1, emb2)
    return (adj, proxy)


# speedup vs baseline: 22.7802x; 1.0257x over previous
import jax
import jax.numpy as jnp
from jax.experimental import pallas as pl

N_ROWS = 4096
N_COLS = 4096
DIM = 64
K = 64
BR = 256
NCHUNK = 32
UNROLL = 16


def _fused_kernel(emb1_ref, emb2_ref, adj_ref, proxy_ref):
    a = emb1_ref[...]
    b = emb2_ref[...]
    logits = jax.lax.dot_general(
        a, b, (((1,), (1,)), ((), ())), preferred_element_type=jnp.float32
    )
    x = jnp.maximum(logits, 0.0)
    xi = jax.lax.bitcast_convert_type(x, jnp.int32)

    fold = xi[:, :128]
    for kchunk in range(1, NCHUNK):
        fold = jnp.maximum(fold, xi[:, kchunk * 128 : (kchunk + 1) * 128])
    lo0 = jnp.min(fold, axis=1, keepdims=True)
    hi0 = jnp.max(fold, axis=1, keepdims=True) + 1

    def count_ge(v):
        return jnp.sum((xi >= v).astype(jnp.int32), axis=1, keepdims=True)

    def step(carry):
        lo, hi, cl, ch = carry
        mid = lo + ((hi - lo) >> 1)
        cnt = count_ge(mid)
        big = cnt >= K
        return (
            jnp.where(big, mid, lo),
            jnp.where(big, hi, mid),
            jnp.where(big, cnt, cl),
            jnp.where(big, ch, cnt),
        )

    def not_done(cy):
        lo, hi, cl, _ = cy
        return jnp.any((cl != K) & (hi - lo > 1))

    carry = (lo0, hi0, count_ge(lo0), jnp.zeros((BR, 1), dtype=jnp.int32))
    for _ in range(UNROLL):
        carry = step(carry)
    carry = jax.lax.while_loop(not_done, step, carry)
    t, thi, n_ge, g = carry

    ge = xi >= t
    gt = xi >= thi
    r = K - g

    col = jax.lax.broadcasted_iota(jnp.int32, (BR, N_COLS), 1)
    surplus = n_ge > K
    clo0 = jnp.where(surplus, -1, N_COLS - 1).astype(jnp.int32)
    chi0 = jnp.full((BR, 1), N_COLS, dtype=jnp.int32)

    def col_body(carry):
        clo, chi = carry
        mid = clo + ((chi - clo) >> 1)
        inb = (ge & ~gt & (col <= mid)).astype(jnp.int32)
        cnt = jnp.sum(inb, axis=1, keepdims=True)
        small = cnt <= r
        return jnp.where(small, mid, clo), jnp.where(small, chi, mid)

    c, _ = jax.lax.while_loop(
        lambda cy: jnp.any(cy[1] - cy[0] > 1), col_body, (clo0, chi0)
    )
    keep = ge & (gt | (col <= c))

    m = jax.lax.bitcast_convert_type(hi0 - 1, jnp.float32)
    e = jnp.exp(x - m)
    ek = jnp.where(keep, e, 0.0)
    s = jnp.sum(ek, axis=1, keepdims=True)
    adj_ref[...] = ek * (1.0 / s)
    proxy_ref[...] = jnp.where(keep, jax.nn.sigmoid(x), 0.5)


@jax.jit
def kernel(emb1, emb2):
    out_shape = [
        jax.ShapeDtypeStruct((N_ROWS, N_COLS), jnp.float32),
        jax.ShapeDtypeStruct((N_ROWS, N_COLS), jnp.float32),
    ]
    adj, proxy = pl.pallas_call(
        _fused_kernel,
        grid=(N_ROWS // BR,),
        in_specs=[
            pl.BlockSpec((BR, DIM), lambda i: (i, 0)),
            pl.BlockSpec((N_COLS, DIM), lambda i: (0, 0)),
        ],
        out_specs=[
            pl.BlockSpec((BR, N_COLS), lambda i: (i, 0)),
            pl.BlockSpec((BR, N_COLS), lambda i: (i, 0)),
        ],
        out_shape=out_shape,
    )(emb1, emb2)
    return (adj, proxy)
